# baseline (speedup 1.0000x reference)
import functools

import jax
import jax.numpy as jnp
from jax import lax
from jax.experimental import pallas as pl
from jax.experimental.pallas import tpu as pltpu
from jax.experimental.pallas import tpu_sc as plsc

MAXLEN = 200
EMBED = 64
LANES = 16
BBLK = 128
NGBUF = 4
NOBUF = 2


def kernel(inputs, token_table, pos_table):
    B, L = inputs.shape
    NC, NS = 2, 16
    NW = NC * NS
    assert B == NW * BBLK and L == MAXLEN
    idx_t = inputs.T.astype(jnp.int32)

    mesh = plsc.VectorSubcoreMesh(
        core_axis_name="c", subcore_axis_name="s", num_cores=NC, num_subcores=NS
    )

    @functools.partial(
        pl.kernel,
        out_type=jax.ShapeDtypeStruct((MAXLEN, EMBED, B), jnp.float32),
        mesh=mesh,
        scratch_types=[
            pltpu.VMEM((MAXLEN, BBLK), jnp.int32),
            pltpu.VMEM((MAXLEN, EMBED), jnp.float32),
            pltpu.VMEM((NGBUF, BBLK, EMBED), jnp.float32),
            pltpu.VMEM((NOBUF, EMBED, BBLK), jnp.float32),
            pltpu.SemaphoreType.DMA((NGBUF,)),
            pltpu.SemaphoreType.DMA((NOBUF,)),
        ],
        compiler_params=pltpu.CompilerParams(
            use_tc_tiling_on_sc=False, needs_layout_passes=False
        ),
    )
    def run(idx_hbm, table_hbm, pos_hbm, out_hbm, idx_v, pos_v, gbuf, obuf, gsem, ssem):
        wid = lax.axis_index("s") * NC + lax.axis_index("c")
        col = wid * BBLK
        pltpu.sync_copy(pos_hbm, pos_v)
        pltpu.sync_copy(idx_hbm.at[:, pl.ds(col, BBLK)], idx_v)

        def issue_gather(t, g):
            pltpu.async_copy(table_hbm.at[idx_v.at[t]], gbuf.at[g], gsem.at[g])

        def wait_gather(t, g):
            pltpu.make_async_copy(
                table_hbm.at[idx_v.at[t]], gbuf.at[g], gsem.at[g]
            ).wait()

        def out_slice(t):
            return out_hbm.at[t, :, pl.ds(col, BBLK)]

        def wait_store(t, o):
            pltpu.make_async_copy(obuf.at[o], out_slice(t), ssem.at[o]).wait()

        iota16 = lax.iota(jnp.int32, 16)

        for g in range(NGBUF - 1):
            issue_gather(g, g)

        def outer(i, carry):
            for g in range(NGBUF):
                t = i * NGBUF + g
                o = t % NOBUF
                wait_gather(t, g)

                @pl.when(t >= NOBUF)
                def _():
                    wait_store(t - NOBUF, o)

                prow = [
                    pos_v[t, pl.ds(j * LANES, LANES)]
                    for j in range(EMBED // LANES)
                ]

                def kblk(k, c):
                    bidx = iota16 + k * LANES
                    for j in range(EMBED // LANES):
                        for e_sub in range(LANES):
                            e = j * LANES + e_sub
                            pvec = jnp.full((LANES,), prow[j][e_sub])
                            eidx = jnp.full((LANES,), e, jnp.int32)
                            v = plsc.load_gather(gbuf.at[g], [bidx, eidx])
                            obuf[o, e, pl.ds(k * LANES, LANES)] = v + pvec
                    return c

                lax.fori_loop(0, BBLK // LANES, kblk, 0)
                pltpu.async_copy(obuf.at[o], out_slice(t), ssem.at[o])

                gp = (g + NGBUF - 1) % NGBUF

                @pl.when(t + NGBUF - 1 < MAXLEN)
                def _():
                    issue_gather(t + NGBUF - 1, gp)
            return carry

        lax.fori_loop(0, MAXLEN // NGBUF, outer, 0)

        for o in range(NOBUF):
            wait_store(MAXLEN - NOBUF + o, o)

    out = run(idx_t, token_table, pos_table)
    return jnp.transpose(out, (2, 0, 1))


# speedup vs baseline: 1.5912x; 1.5912x over previous
import functools

import jax
import jax.numpy as jnp
from jax import lax
from jax.experimental import pallas as pl
from jax.experimental.pallas import tpu as pltpu
from jax.experimental.pallas import tpu_sc as plsc

MAXLEN = 200
EMBED = 64
LANES = 16
BBLK = 128
NGBUF = 4
NOBUF = 2
BPAD = 129
UNROLL_B = 2


def kernel(inputs, token_table, pos_table):
    B, L = inputs.shape
    NC, NS = 2, 16
    NW = NC * NS
    assert B == NW * BBLK and L == MAXLEN
    idx_t = inputs.T.astype(jnp.int32)

    mesh = plsc.VectorSubcoreMesh(
        core_axis_name="c", subcore_axis_name="s", num_cores=NC, num_subcores=NS
    )

    @functools.partial(
        pl.kernel,
        out_type=jax.ShapeDtypeStruct((MAXLEN, EMBED, B), jnp.float32),
        mesh=mesh,
        scratch_types=[
            pltpu.VMEM((MAXLEN, BBLK), jnp.int32),
            pltpu.VMEM((MAXLEN, EMBED), jnp.float32),
            pltpu.VMEM((NGBUF, BBLK, EMBED), jnp.float32),
            pltpu.VMEM((NOBUF, EMBED, BPAD), jnp.float32),
            pltpu.SemaphoreType.DMA((NGBUF,)),
            pltpu.SemaphoreType.DMA((NOBUF,)),
        ],
        compiler_params=pltpu.CompilerParams(
            use_tc_tiling_on_sc=False, needs_layout_passes=False
        ),
    )
    def run(idx_hbm, table_hbm, pos_hbm, out_hbm, idx_v, pos_v, gbuf, obuf, gsem, ssem):
        wid = lax.axis_index("s") * NC + lax.axis_index("c")
        col = wid * BBLK
        pltpu.sync_copy(pos_hbm, pos_v)
        pltpu.sync_copy(idx_hbm.at[:, pl.ds(col, BBLK)], idx_v)

        def issue_gather(t, g):
            pltpu.async_copy(table_hbm.at[idx_v.at[t]], gbuf.at[g], gsem.at[g])

        def wait_gather(t, g):
            pltpu.make_async_copy(
                table_hbm.at[idx_v.at[t]], gbuf.at[g], gsem.at[g]
            ).wait()

        def out_slice(t):
            return out_hbm.at[t, :, pl.ds(col, BBLK)]

        def store_src(o):
            return obuf.at[o, :, pl.ds(0, BBLK)]

        def wait_store(t, o):
            pltpu.make_async_copy(store_src(o), out_slice(t), ssem.at[o]).wait()

        iota16 = lax.iota(jnp.int32, 16)

        for g in range(NGBUF - 1):
            issue_gather(g, g)

        def outer(i, carry):
            for g in range(NGBUF):
                t = i * NGBUF + g
                o = t % NOBUF
                wait_gather(t, g)

                @pl.when(t >= NOBUF)
                def _():
                    wait_store(t - NOBUF, o)

                prow = [
                    pos_v[t, pl.ds(j * LANES, LANES)]
                    for j in range(EMBED // LANES)
                ]
                eidx = [iota16 + j * LANES for j in range(EMBED // LANES)]

                def bblk(i2, c):
                    for u in range(UNROLL_B):
                        b = i2 * UNROLL_B + u
                        bvec = jnp.full((LANES,), b, jnp.int32)
                        for j in range(EMBED // LANES):
                            v = gbuf[g, b, pl.ds(j * LANES, LANES)] + prow[j]
                            plsc.store_scatter(obuf.at[o], [eidx[j], bvec], v)
                    return c

                lax.fori_loop(0, BBLK // UNROLL_B, bblk, 0)
                pltpu.async_copy(store_src(o), out_slice(t), ssem.at[o])

                gp = (g + NGBUF - 1) % NGBUF

                @pl.when(t + NGBUF - 1 < MAXLEN)
                def _():
                    issue_gather(t + NGBUF - 1, gp)
            return carry

        lax.fori_loop(0, MAXLEN // NGBUF, outer, 0)

        for o in range(NOBUF):
            wait_store(MAXLEN - NOBUF + o, o)

    out = run(idx_t, token_table, pos_table)
    return jnp.transpose(out, (2, 0, 1))


# speedup vs baseline: 1.9783x; 1.2433x over previous
import functools

import jax
import jax.numpy as jnp
from jax import lax
from jax.experimental import pallas as pl
from jax.experimental.pallas import tpu as pltpu
from jax.experimental.pallas import tpu_sc as plsc

MAXLEN = 200
EMBED = 64
LANES = 16
BBLK = 128
NGBUF = 4
NOBUF = 2
BPAD = 129
UNROLL_B = 4


def kernel(inputs, token_table, pos_table):
    B, L = inputs.shape
    NC, NS = 2, 16
    NW = NC * NS
    assert B == NW * BBLK and L == MAXLEN
    idx_t = inputs.T.astype(jnp.int32)

    mesh = plsc.VectorSubcoreMesh(
        core_axis_name="c", subcore_axis_name="s", num_cores=NC, num_subcores=NS
    )

    @functools.partial(
        pl.kernel,
        out_type=jax.ShapeDtypeStruct((MAXLEN, EMBED, B), jnp.float32),
        mesh=mesh,
        scratch_types=[
            pltpu.VMEM((MAXLEN, BBLK), jnp.int32),
            pltpu.VMEM((MAXLEN, EMBED), jnp.float32),
            pltpu.VMEM((NGBUF, BBLK, EMBED), jnp.float32),
            pltpu.VMEM((NOBUF, EMBED, BPAD), jnp.float32),
            pltpu.SemaphoreType.DMA((NGBUF,)),
            pltpu.SemaphoreType.DMA((NOBUF,)),
        ],
        compiler_params=pltpu.CompilerParams(
            use_tc_tiling_on_sc=False, needs_layout_passes=False
        ),
    )
    def run(idx_hbm, table_hbm, pos_hbm, out_hbm, idx_v, pos_v, gbuf, obuf, gsem, ssem):
        wid = lax.axis_index("s") * NC + lax.axis_index("c")
        col = wid * BBLK
        pltpu.sync_copy(pos_hbm, pos_v)
        pltpu.sync_copy(idx_hbm.at[:, pl.ds(col, BBLK)], idx_v)

        def issue_gather(t, g):
            pltpu.async_copy(table_hbm.at[idx_v.at[t]], gbuf.at[g], gsem.at[g])

        def wait_gather(t, g):
            pltpu.make_async_copy(
                table_hbm.at[idx_v.at[t]], gbuf.at[g], gsem.at[g]
            ).wait()

        def out_slice(t):
            return out_hbm.at[t, :, pl.ds(col, BBLK)]

        def store_src(o):
            return obuf.at[o, :, pl.ds(0, BBLK)]

        def wait_store(t, o):
            pltpu.make_async_copy(store_src(o), out_slice(t), ssem.at[o]).wait()

        iota16 = lax.iota(jnp.int32, 16)

        for g in range(NGBUF - 1):
            issue_gather(g, g)

        def outer(i, carry):
            for g in range(NGBUF):
                t = i * NGBUF + g
                o = t % NOBUF
                wait_gather(t, g)

                @pl.when(t >= NOBUF)
                def _():
                    wait_store(t - NOBUF, o)

                prow = [
                    pos_v[t, pl.ds(j * LANES, LANES)]
                    for j in range(EMBED // LANES)
                ]
                eidx = [iota16 + j * LANES for j in range(EMBED // LANES)]

                @plsc.parallel_loop(0, BBLK // UNROLL_B, unroll=2)
                def _(i2):
                    for u in range(UNROLL_B):
                        b = i2 * UNROLL_B + u
                        bvec = jnp.full((LANES,), b, jnp.int32)
                        for j in range(EMBED // LANES):
                            v = gbuf[g, b, pl.ds(j * LANES, LANES)] + prow[j]
                            plsc.store_scatter(obuf.at[o], [eidx[j], bvec], v)
                pltpu.async_copy(store_src(o), out_slice(t), ssem.at[o])

                gp = (g + NGBUF - 1) % NGBUF

                @pl.when(t + NGBUF - 1 < MAXLEN)
                def _():
                    issue_gather(t + NGBUF - 1, gp)
            return carry

        lax.fori_loop(0, MAXLEN // NGBUF, outer, 0)

        for o in range(NOBUF):
            wait_store(MAXLEN - NOBUF + o, o)

    out = run(idx_t, token_table, pos_table)
    return jnp.transpose(out, (2, 0, 1))


# speedup vs baseline: 2.5113x; 1.2694x over previous
import functools

import jax
import jax.numpy as jnp
from jax import lax
from jax.experimental import pallas as pl
from jax.experimental.pallas import tpu as pltpu
from jax.experimental.pallas import tpu_sc as plsc

MAXLEN = 200
EMBED = 64
LANES = 16
BBLK = 128
NGBUF = 4
NOBUF = 2
BPAD = 129
UNROLL_B = 4


def kernel(inputs, token_table, pos_table):
    B, L = inputs.shape
    NC, NS = 2, 16
    NW = NC * NS
    assert B == NW * BBLK and L == MAXLEN
    idx_t = inputs.T.astype(jnp.int32)

    mesh = plsc.VectorSubcoreMesh(
        core_axis_name="c", subcore_axis_name="s", num_cores=NC, num_subcores=NS
    )

    @functools.partial(
        pl.kernel,
        out_type=jax.ShapeDtypeStruct((MAXLEN, 8, B // BBLK, EMBED // 8, BBLK), jnp.float32),
        mesh=mesh,
        scratch_types=[
            pltpu.VMEM((MAXLEN, BBLK), jnp.int32),
            pltpu.VMEM((MAXLEN, EMBED), jnp.float32),
            pltpu.VMEM((NGBUF, BBLK, EMBED), jnp.float32),
            pltpu.VMEM((NOBUF, 8, EMBED // 8, BPAD), jnp.float32),
            pltpu.SemaphoreType.DMA((NGBUF,)),
            pltpu.SemaphoreType.DMA((NOBUF,)),
        ],
        compiler_params=pltpu.CompilerParams(
            use_tc_tiling_on_sc=False, needs_layout_passes=False
        ),
    )
    def run(idx_hbm, table_hbm, pos_hbm, out_hbm, idx_v, pos_v, gbuf, obuf, gsem, ssem):
        wid = lax.axis_index("s") * NC + lax.axis_index("c")
        col = wid * BBLK
        pltpu.sync_copy(pos_hbm, pos_v)
        pltpu.sync_copy(idx_hbm.at[:, pl.ds(col, BBLK)], idx_v)

        def issue_gather(t, g):
            pltpu.async_copy(table_hbm.at[idx_v.at[t]], gbuf.at[g], gsem.at[g])

        def wait_gather(t, g):
            pltpu.make_async_copy(
                table_hbm.at[idx_v.at[t]], gbuf.at[g], gsem.at[g]
            ).wait()

        def out_slice(t):
            return out_hbm.at[t, :, wid, :, :]

        def store_src(o):
            return obuf.at[o, :, :, pl.ds(0, BBLK)]

        def wait_store(t, o):
            pltpu.make_async_copy(store_src(o), out_slice(t), ssem.at[o]).wait()

        iota16 = lax.iota(jnp.int32, 16)

        for g in range(NGBUF - 1):
            issue_gather(g, g)

        def outer(i, carry):
            for g in range(NGBUF):
                t = i * NGBUF + g
                o = t % NOBUF
                wait_gather(t, g)

                @pl.when(t >= NOBUF)
                def _():
                    wait_store(t - NOBUF, o)

                prow = [
                    pos_v[t, pl.ds(j * LANES, LANES)]
                    for j in range(EMBED // LANES)
                ]
                eidx = [iota16 + j * LANES for j in range(EMBED // LANES)]
                ehi = [e >> 3 for e in eidx]
                elo = [e & 7 for e in eidx]

                @plsc.parallel_loop(0, BBLK // UNROLL_B, unroll=2)
                def _(i2):
                    for u in range(UNROLL_B):
                        b = i2 * UNROLL_B + u
                        bvec = jnp.full((LANES,), b, jnp.int32)
                        for j in range(EMBED // LANES):
                            v = gbuf[g, b, pl.ds(j * LANES, LANES)] + prow[j]
                            plsc.store_scatter(
                                obuf.at[o], [ehi[j], elo[j], bvec], v
                            )
                pltpu.async_copy(store_src(o), out_slice(t), ssem.at[o])

                gp = (g + NGBUF - 1) % NGBUF

                @pl.when(t + NGBUF - 1 < MAXLEN)
                def _():
                    issue_gather(t + NGBUF - 1, gp)
            return carry

        lax.fori_loop(0, MAXLEN // NGBUF, outer, 0)

        for o in range(NOBUF):
            wait_store(MAXLEN - NOBUF + o, o)

    out = run(idx_t, token_table, pos_table)
    out = jnp.transpose(out, (0, 1, 3, 2, 4)).reshape(MAXLEN, EMBED, B)
    return jnp.transpose(out, (2, 0, 1))
